# baseline (speedup 1.0000x reference)
import functools

import jax
import jax.numpy as jnp
from jax.experimental import pallas as pl
from jax.experimental.pallas import tpu as pltpu

E = 64
TOPK = 8
LOSS_COEF = 0.01
BM = 1024
BC = 256


def _gating_kernel(x_ref, w_in_ref, idx_ref, w_ref, pi_ref, cnt_ref, aux_ref,
                   lg_ref, *, total_tokens):
    i = pl.program_id(0)
    nsteps = pl.num_programs(0)

    @pl.when(i == 0)
    def _init():
        lg_ref[...] = jnp.zeros_like(lg_ref)
        pi_ref[...] = jnp.zeros_like(pi_ref)
        cnt_ref[...] = jnp.zeros_like(cnt_ref)

    flag = (i > 0).astype(jnp.float32)
    pi_part = jnp.zeros((1, E), jnp.float32)
    cnt_part = jnp.zeros((1, E), jnp.float32)
    for c in range(BM // BC):
        lg = lg_ref[c * BC:(c + 1) * BC, :]
        m = jnp.max(lg, axis=-1, keepdims=True)
        ex = jnp.exp(lg - m)
        denom = jnp.sum(ex, axis=-1, keepdims=True)
        scores = ex / denom

        iota = jax.lax.broadcasted_iota(jnp.int32, scores.shape, 1)
        sbits = jax.lax.bitcast_convert_type(scores, jnp.int32)
        work = jax.lax.bitcast_convert_type(
            jnp.bitwise_or(jnp.bitwise_and(sbits, -64), (E - 1) - iota),
            jnp.float32)
        vals = []
        idxs = []
        for _ in range(TOPK):
            mk = jnp.max(work, axis=-1, keepdims=True)
            work = jnp.where(work == mk, -1.0, work)
            mbits = jax.lax.bitcast_convert_type(mk, jnp.int32)
            idxs.append((E - 1) - jnp.bitwise_and(mbits, E - 1))
            vals.append(jax.lax.bitcast_convert_type(
                jnp.bitwise_and(mbits, -64), jnp.float32))
        topv = jnp.concatenate(vals, axis=-1)
        topi = jnp.concatenate(idxs, axis=-1)
        topv = topv / jnp.sum(topv, axis=-1, keepdims=True)

        idx_ref[c * BC:(c + 1) * BC, :] = topi.astype(jnp.int32)
        w_ref[c * BC:(c + 1) * BC, :] = topv

        sel = (work < 0.0).astype(jnp.float32)
        pi_part += jnp.sum(scores, axis=0, keepdims=True)
        cnt_part += jnp.sum(sel, axis=0, keepdims=True)

    pi_ref[...] += pi_part * flag
    cnt_ref[...] += cnt_part * flag

    x = x_ref[...]
    w = w_in_ref[...]
    lg_ref[...] = jax.lax.dot_general(
        x, w, (((1,), (1,)), ((), ())),
        preferred_element_type=jnp.float32)

    @pl.when(i == nsteps - 1)
    def _finish():
        scale = LOSS_COEF * E / (float(total_tokens) ** 2 * TOPK)
        aux = jnp.sum(pi_ref[...] * cnt_ref[...]) * scale
        aux_ref[...] = jnp.full((1, 1), aux, dtype=jnp.float32)


def kernel(x, W):
    Bd, Nd, R = x.shape
    T = Bd * Nd
    flat_x = x.reshape(T, R)
    nblk = T // BM

    out_shapes = (
        jax.ShapeDtypeStruct((T, TOPK), jnp.int32),
        jax.ShapeDtypeStruct((T, TOPK), jnp.float32),
        jax.ShapeDtypeStruct((1, E), jnp.float32),
        jax.ShapeDtypeStruct((1, E), jnp.float32),
        jax.ShapeDtypeStruct((1, 1), jnp.float32),
    )
    idx, w, _pi, _cnt, aux = pl.pallas_call(
        functools.partial(_gating_kernel, total_tokens=T),
        grid=(nblk + 1,),
        in_specs=[
            pl.BlockSpec((BM, R), lambda i: (jnp.minimum(i, nblk - 1), 0)),
            pl.BlockSpec((E, R), lambda i: (0, 0)),
        ],
        out_specs=[
            pl.BlockSpec((BM, TOPK), lambda i: (jnp.maximum(i - 1, 0), 0)),
            pl.BlockSpec((BM, TOPK), lambda i: (jnp.maximum(i - 1, 0), 0)),
            pl.BlockSpec((1, E), lambda i: (0, 0)),
            pl.BlockSpec((1, E), lambda i: (0, 0)),
            pl.BlockSpec((1, 1), lambda i: (0, 0)),
        ],
        out_shape=out_shapes,
        scratch_shapes=[pltpu.VMEM((BM, E), jnp.float32)],
        compiler_params=pltpu.CompilerParams(
            dimension_semantics=("arbitrary",),
        ),
    )(flat_x, W)

    return (idx.reshape(Bd, Nd, TOPK), w.reshape(Bd, Nd, TOPK), aux[0, 0])


# speedup vs baseline: 1.0056x; 1.0056x over previous
import functools

import jax
import jax.numpy as jnp
from jax.experimental import pallas as pl
from jax.experimental.pallas import tpu as pltpu

E = 64
TOPK = 8
LOSS_COEF = 0.01
BM = 1024
BC = 256


def _gating_kernel(x1_ref, x2_ref, w_in_ref, idx_ref, w_ref, pi_ref, cnt_ref,
                   aux_ref, lg_ref, *, total_tokens):
    i = pl.program_id(0)
    nsteps = pl.num_programs(0)

    @pl.when(i == 0)
    def _init():
        lg_ref[...] = jnp.zeros_like(lg_ref)
        pi_ref[...] = jnp.zeros_like(pi_ref)
        cnt_ref[...] = jnp.zeros_like(cnt_ref)

    flag = (i > 0).astype(jnp.float32)
    pi_part = jnp.zeros((1, E), jnp.float32)
    cnt_part = jnp.zeros((1, E), jnp.float32)
    for c in range(BM // BC):
        lg = lg_ref[c * BC:(c + 1) * BC, :]
        m = jnp.max(lg, axis=-1, keepdims=True)
        ex = jnp.exp(lg - m)
        denom = jnp.sum(ex, axis=-1, keepdims=True)
        scores = ex / denom

        iota = jax.lax.broadcasted_iota(jnp.int32, scores.shape, 1)
        sbits = jax.lax.bitcast_convert_type(scores, jnp.int32)
        work = jax.lax.bitcast_convert_type(
            jnp.bitwise_or(jnp.bitwise_and(sbits, -64), (E - 1) - iota),
            jnp.float32)
        vals = []
        idxs = []
        for _ in range(TOPK):
            mk = jnp.max(work, axis=-1, keepdims=True)
            work = jnp.where(work == mk, -1.0, work)
            mbits = jax.lax.bitcast_convert_type(mk, jnp.int32)
            idxs.append((E - 1) - jnp.bitwise_and(mbits, E - 1))
            vals.append(jax.lax.bitcast_convert_type(
                jnp.bitwise_and(mbits, -64), jnp.float32))
        topv = jnp.concatenate(vals, axis=-1)
        topi = jnp.concatenate(idxs, axis=-1)
        topv = topv / jnp.sum(topv, axis=-1, keepdims=True)

        idx_ref[c * BC:(c + 1) * BC, :] = topi.astype(jnp.int32)
        w_ref[c * BC:(c + 1) * BC, :] = topv

        sel = (work < 0.0).astype(jnp.float32)
        pi_part += jnp.sum(scores, axis=0, keepdims=True)
        cnt_part += jnp.sum(sel, axis=0, keepdims=True)

    pi_ref[...] += pi_part * flag
    cnt_ref[...] += cnt_part * flag

    w = w_in_ref[...]
    h = w.shape[1] // 2
    lg_ref[...] = (
        jax.lax.dot_general(x1_ref[...], w[:, :h], (((1,), (1,)), ((), ())),
                            preferred_element_type=jnp.float32)
        + jax.lax.dot_general(x2_ref[...], w[:, h:], (((1,), (1,)), ((), ())),
                              preferred_element_type=jnp.float32))

    @pl.when(i == nsteps - 1)
    def _finish():
        scale = LOSS_COEF * E / (float(total_tokens) ** 2 * TOPK)
        aux = jnp.sum(pi_ref[...] * cnt_ref[...]) * scale
        aux_ref[...] = jnp.full((1, 1), aux, dtype=jnp.float32)


def kernel(x, W):
    Bd, Nd, R = x.shape
    T = Bd * Nd
    flat_x = x.reshape(T, R)
    nblk = T // BM

    out_shapes = (
        jax.ShapeDtypeStruct((T, TOPK), jnp.int32),
        jax.ShapeDtypeStruct((T, TOPK), jnp.float32),
        jax.ShapeDtypeStruct((1, E), jnp.float32),
        jax.ShapeDtypeStruct((1, E), jnp.float32),
        jax.ShapeDtypeStruct((1, 1), jnp.float32),
    )
    idx, w, _pi, _cnt, aux = pl.pallas_call(
        functools.partial(_gating_kernel, total_tokens=T),
        grid=(nblk + 1,),
        in_specs=[
            pl.BlockSpec((BM, R // 2), lambda i: (jnp.minimum(i, nblk - 1), 0)),
            pl.BlockSpec((BM, R // 2), lambda i: (jnp.minimum(i, nblk - 1), 1)),
            pl.BlockSpec((E, R), lambda i: (0, 0)),
        ],
        out_specs=[
            pl.BlockSpec((BM, TOPK), lambda i: (jnp.maximum(i - 1, 0), 0)),
            pl.BlockSpec((BM, TOPK), lambda i: (jnp.maximum(i - 1, 0), 0)),
            pl.BlockSpec((1, E), lambda i: (0, 0)),
            pl.BlockSpec((1, E), lambda i: (0, 0)),
            pl.BlockSpec((1, 1), lambda i: (0, 0)),
        ],
        out_shape=out_shapes,
        scratch_shapes=[pltpu.VMEM((BM, E), jnp.float32)],
        compiler_params=pltpu.CompilerParams(
            dimension_semantics=("arbitrary",),
        ),
    )(flat_x, flat_x, W)

    return (idx.reshape(Bd, Nd, TOPK), w.reshape(Bd, Nd, TOPK), aux[0, 0])
